# baseline (speedup 1.0000x reference)
import functools

import jax
import jax.numpy as jnp
from jax import lax
from jax.experimental import pallas as pl
from jax.experimental.pallas import tpu as pltpu
from jax.experimental.pallas import tpu_sc as plsc

NUM_SEG = 1024
D = 128
N_ROWS = 320000
NC = 2
NS = 16
NW = NC * NS
RPW = N_ROWS // NW
CHUNK = 80
NCHUNK = RPW // CHUNK
SEG_PER_TILE = NUM_SEG // NS
NBUF = 5
NBKT = 16


def _sc_partials(data, ids):
    mesh = plsc.VectorSubcoreMesh(core_axis_name="c", subcore_axis_name="s")

    @functools.partial(
        pl.kernel,
        out_type=jax.ShapeDtypeStruct((NC, NUM_SEG, D), jnp.float32),
        mesh=mesh,
        scratch_types=[
            pltpu.VMEM((NBUF, CHUNK, D), jnp.float32),
            pltpu.VMEM((NCHUNK, CHUNK), jnp.int32),
            pltpu.VMEM((SEG_PER_TILE, D), jnp.float32),
            pltpu.VMEM_SHARED((NUM_SEG, D), jnp.float32),
            pltpu.VMEM((NBKT, D), jnp.float32),
            pltpu.VMEM((NBKT,), jnp.int32),
            [pltpu.SemaphoreType.DMA] * NBUF,
        ],
    )
    def body(data_hbm, ids_hbm, out_hbm, rowbuf, idsbuf, zbuf, acc,
             bucket, idxstg, sems):
        cid = lax.axis_index("c")
        sid = lax.axis_index("s")
        wid = cid * NS + sid
        base_row = wid * RPW

        def gather(ch, b):
            return pltpu.make_async_copy(
                data_hbm.at[pl.ds(base_row + ch * CHUNK, CHUNK)],
                rowbuf.at[b],
                sems[b],
            )

        for b in range(NBUF):
            gather(b, b).start()
        pltpu.sync_copy(ids_hbm.at[wid], idsbuf)

        zero = jnp.zeros((16,), jnp.float32)
        iota16 = lax.iota(jnp.int32, 16)

        def zero_body(i, carry):
            for j in range(D // 16):
                zbuf[i, pl.ds(j * 16, 16)] = zero
            return carry

        lax.fori_loop(0, SEG_PER_TILE, zero_body, 0)
        pltpu.sync_copy(zbuf, acc.at[pl.ds(sid * SEG_PER_TILE, SEG_PER_TILE)])
        for i in range(NBKT):
            for j in range(D // 16):
                bucket[i, pl.ds(j * 16, 16)] = zero
        plsc.subcore_barrier()

        def group_body(g, carry):
            ch0 = g * NBUF
            for b in range(NBUF):
                ch = ch0 + b
                gather(ch, b).wait()
                first = idsbuf[ch, pl.ds(0, 16)][0]
                span = idsbuf[ch, pl.ds(CHUNK - 16, 16)][15] - first

                @pl.when(span < NBKT)
                def _fast():
                    def row_body(t, c2):
                        idvec = idsbuf[ch, pl.ds(t * 16, 16)]
                        for l in range(16):
                            slot = idvec[l] & (NBKT - 1)
                            r = t * 16 + l
                            for j in range(D // 16):
                                v = rowbuf[b, r, pl.ds(j * 16, 16)]
                                plsc.addupdate(
                                    bucket.at[slot, pl.ds(j * 16, 16)], v)
                        return c2

                    lax.fori_loop(0, CHUNK // 16, row_body, 0)
                    cand = (first & ~(NBKT - 1)) + iota16
                    idx = jnp.minimum(
                        cand + jnp.where(cand < first, NBKT, 0),
                        NUM_SEG - 1)
                    idxstg[pl.ds(0, NBKT)] = idx
                    pltpu.sync_copy(bucket, acc.at[idxstg], add=True)

                    def rezero(i, c2):
                        for j in range(D // 16):
                            bucket[i, pl.ds(j * 16, 16)] = zero
                        return c2

                    lax.fori_loop(0, NBKT, rezero, 0)

                @pl.when(span >= NBKT)
                def _slow():
                    pltpu.sync_copy(rowbuf.at[b], acc.at[idsbuf.at[ch]],
                                    add=True)

                @pl.when(ch + NBUF < NCHUNK)
                def _():
                    gather(ch + NBUF, b).start()

            return carry

        lax.fori_loop(0, NCHUNK // NBUF, group_body, 0)
        plsc.subcore_barrier()

        pltpu.sync_copy(
            acc.at[pl.ds(sid * SEG_PER_TILE, SEG_PER_TILE)],
            out_hbm.at[cid].at[pl.ds(sid * SEG_PER_TILE, SEG_PER_TILE)],
        )

    return body(data, ids)


def _combine_body(p_ref, o_ref):
    o_ref[...] = p_ref[0] + p_ref[1]


_combine = pl.pallas_call(
    _combine_body,
    out_shape=jax.ShapeDtypeStruct((NUM_SEG, D), jnp.float32),
)


def kernel(data, segment_ids):
    ids = segment_ids.astype(jnp.int32).reshape(NW, NCHUNK, CHUNK)
    partials = _sc_partials(data, ids)
    return _combine(partials)


# speedup vs baseline: 1.8342x; 1.8342x over previous
import functools

import jax
import jax.numpy as jnp
from jax import lax
from jax.experimental import pallas as pl
from jax.experimental.pallas import tpu as pltpu
from jax.experimental.pallas import tpu_sc as plsc

NUM_SEG = 1024
D = 128
N_ROWS = 320000
NC = 2
NS = 16
NW = NC * NS
RPW = N_ROWS // NW
CHUNK = 80
NCHUNK = RPW // CHUNK
SEG_PER_TILE = NUM_SEG // NS
NBUF = 5
NBKT = 16


def _sc_partials(data, ids):
    mesh = plsc.VectorSubcoreMesh(core_axis_name="c", subcore_axis_name="s")

    @functools.partial(
        pl.kernel,
        out_type=jax.ShapeDtypeStruct((NC, NUM_SEG, D), jnp.float32),
        mesh=mesh,
        scratch_types=[
            pltpu.VMEM((NBUF, CHUNK, D), jnp.float32),
            pltpu.VMEM((NCHUNK, CHUNK), jnp.int32),
            pltpu.VMEM((SEG_PER_TILE, D), jnp.float32),
            pltpu.VMEM_SHARED((NUM_SEG, D), jnp.float32),
            pltpu.VMEM((NBKT, D), jnp.float32),
            pltpu.VMEM((NBKT,), jnp.int32),
            [pltpu.SemaphoreType.DMA] * NBUF,
        ],
    )
    def body(data_hbm, ids_hbm, out_hbm, rowbuf, idsbuf, zbuf, acc,
             bucket, idxstg, sems):
        cid = lax.axis_index("c")
        sid = lax.axis_index("s")
        wid = cid * NS + sid
        base_row = wid * RPW

        def gather(ch, b):
            return pltpu.make_async_copy(
                data_hbm.at[pl.ds(base_row + ch * CHUNK, CHUNK)],
                rowbuf.at[b],
                sems[b],
            )

        for b in range(NBUF):
            gather(b, b).start()
        pltpu.sync_copy(ids_hbm.at[wid], idsbuf)

        zero = jnp.zeros((16,), jnp.float32)
        iota16 = lax.iota(jnp.int32, 16)

        def zero_body(i, carry):
            for j in range(D // 16):
                zbuf[i, pl.ds(j * 16, 16)] = zero
            return carry

        lax.fori_loop(0, SEG_PER_TILE, zero_body, 0)
        pltpu.sync_copy(zbuf, acc.at[pl.ds(sid * SEG_PER_TILE, SEG_PER_TILE)])
        for i in range(NBKT):
            for j in range(D // 16):
                bucket[i, pl.ds(j * 16, 16)] = zero
        plsc.subcore_barrier()

        def group_body(g, carry):
            ch0 = g * NBUF
            for b in range(NBUF):
                ch = ch0 + b
                gather(ch, b).wait()
                first = idsbuf[ch, pl.ds(0, 16)][0]
                span = idsbuf[ch, pl.ds(CHUNK - 16, 16)][15] - first

                @pl.when(span < NBKT)
                def _fast():
                    def row_body(t, c2):
                        idvec = idsbuf[ch, pl.ds(t * 16, 16)]
                        u_first = idvec[0]
                        u_last = idvec[15]

                        @pl.when(u_first == u_last)
                        def _uniform():
                            accs = [rowbuf[b, t * 16, pl.ds(j * 16, 16)]
                                    for j in range(D // 16)]
                            for l in range(1, 16):
                                for j in range(D // 16):
                                    accs[j] = accs[j] + rowbuf[
                                        b, t * 16 + l, pl.ds(j * 16, 16)]
                            slot = u_first & (NBKT - 1)
                            for j in range(D // 16):
                                plsc.addupdate(
                                    bucket.at[slot, pl.ds(j * 16, 16)],
                                    accs[j])

                        @pl.when(u_first != u_last)
                        def _mixed():
                            for l in range(16):
                                slot = idvec[l] & (NBKT - 1)
                                r = t * 16 + l
                                for j in range(D // 16):
                                    v = rowbuf[b, r, pl.ds(j * 16, 16)]
                                    plsc.addupdate(
                                        bucket.at[slot, pl.ds(j * 16, 16)],
                                        v)

                        return c2

                    lax.fori_loop(0, CHUNK // 16, row_body, 0)
                    cand = (first & ~(NBKT - 1)) + iota16
                    idx = jnp.minimum(
                        cand + jnp.where(cand < first, NBKT, 0),
                        NUM_SEG - 1)
                    idxstg[pl.ds(0, NBKT)] = idx
                    pltpu.sync_copy(bucket, acc.at[idxstg], add=True)

                    def rezero(i, c2):
                        for j in range(D // 16):
                            bucket[i, pl.ds(j * 16, 16)] = zero
                        return c2

                    lax.fori_loop(0, NBKT, rezero, 0)

                @pl.when(span >= NBKT)
                def _slow():
                    pltpu.sync_copy(rowbuf.at[b], acc.at[idsbuf.at[ch]],
                                    add=True)

                @pl.when(ch + NBUF < NCHUNK)
                def _():
                    gather(ch + NBUF, b).start()

            return carry

        lax.fori_loop(0, NCHUNK // NBUF, group_body, 0)
        plsc.subcore_barrier()

        pltpu.sync_copy(
            acc.at[pl.ds(sid * SEG_PER_TILE, SEG_PER_TILE)],
            out_hbm.at[cid].at[pl.ds(sid * SEG_PER_TILE, SEG_PER_TILE)],
        )

    return body(data, ids)


def _combine_body(p_ref, o_ref):
    o_ref[...] = p_ref[0] + p_ref[1]


_combine = pl.pallas_call(
    _combine_body,
    out_shape=jax.ShapeDtypeStruct((NUM_SEG, D), jnp.float32),
)


def kernel(data, segment_ids):
    ids = segment_ids.astype(jnp.int32).reshape(NW, NCHUNK, CHUNK)
    partials = _sc_partials(data, ids)
    return _combine(partials)


# speedup vs baseline: 2.2746x; 1.2401x over previous
import functools

import jax
import jax.numpy as jnp
from jax import lax
from jax.experimental import pallas as pl
from jax.experimental.pallas import tpu as pltpu
from jax.experimental.pallas import tpu_sc as plsc

NUM_SEG = 1024
D = 128
N_ROWS = 320000
NC = 2
NS = 16
NW = NC * NS
RPW = N_ROWS // NW
CHUNK = 80
NCHUNK = RPW // CHUNK
SEG_PER_TILE = NUM_SEG // NS
NBUF = 5
LAG = 2


def _sc_partials(data, ids):
    mesh = plsc.VectorSubcoreMesh(core_axis_name="c", subcore_axis_name="s")

    @functools.partial(
        pl.kernel,
        out_type=jax.ShapeDtypeStruct((NC, NUM_SEG, D), jnp.float32),
        mesh=mesh,
        scratch_types=[
            pltpu.VMEM((NBUF, CHUNK, D), jnp.float32),
            pltpu.VMEM((NCHUNK, CHUNK), jnp.int32),
            pltpu.VMEM((SEG_PER_TILE, D), jnp.float32),
            pltpu.VMEM_SHARED((NUM_SEG, D), jnp.float32),
            [pltpu.SemaphoreType.DMA] * NBUF,
            [pltpu.SemaphoreType.DMA] * NBUF,
        ],
    )
    def body(data_hbm, ids_hbm, out_hbm, rowbuf, idsbuf, zbuf, acc,
             gsems, ssems):
        cid = lax.axis_index("c")
        sid = lax.axis_index("s")
        wid = cid * NS + sid
        base_row = wid * RPW

        def gather(ch, b):
            return pltpu.make_async_copy(
                data_hbm.at[pl.ds(base_row + ch * CHUNK, CHUNK)],
                rowbuf.at[b],
                gsems[b],
            )

        def scatter(ch, b):
            return pltpu.make_async_copy(
                rowbuf.at[b], acc.at[idsbuf.at[ch]], ssems[b])

        for b in range(NBUF):
            gather(b, b).start()
        pltpu.sync_copy(ids_hbm.at[wid], idsbuf)

        zero = jnp.zeros((16,), jnp.float32)

        def zero_body(i, carry):
            for j in range(D // 16):
                zbuf[i, pl.ds(j * 16, 16)] = zero
            return carry

        lax.fori_loop(0, SEG_PER_TILE, zero_body, 0)
        pltpu.sync_copy(zbuf, acc.at[pl.ds(sid * SEG_PER_TILE, SEG_PER_TILE)])
        plsc.subcore_barrier()

        def group_step(g, carry):
            for b in range(NBUF):
                ch = g * NBUF + b
                gather(ch, b).wait()
                scatter(ch, b).start(add=True)
                bo = (b - LAG) % NBUF
                cho = ch - LAG

                @pl.when(cho >= 0)
                def _():
                    scatter(cho, bo).wait()

                    @pl.when(cho + NBUF < NCHUNK)
                    def _():
                        gather(cho + NBUF, bo).start()

            return carry

        lax.fori_loop(0, NCHUNK // NBUF, group_step, 0)
        for t in range(LAG):
            ch = NCHUNK - LAG + t
            scatter(ch, ch % NBUF).wait()
        plsc.subcore_barrier()

        pltpu.sync_copy(
            acc.at[pl.ds(sid * SEG_PER_TILE, SEG_PER_TILE)],
            out_hbm.at[cid].at[pl.ds(sid * SEG_PER_TILE, SEG_PER_TILE)],
        )

    return body(data, ids)


def _combine_body(p_ref, o_ref):
    o_ref[...] = p_ref[0] + p_ref[1]


_combine = pl.pallas_call(
    _combine_body,
    out_shape=jax.ShapeDtypeStruct((NUM_SEG, D), jnp.float32),
)


def kernel(data, segment_ids):
    ids = segment_ids.astype(jnp.int32).reshape(NW, NCHUNK, CHUNK)
    partials = _sc_partials(data, ids)
    return _combine(partials)
